# baseline (speedup 1.0000x reference)
import functools

import jax
import jax.numpy as jnp
from jax import lax
from jax.experimental import pallas as pl
from jax.experimental.pallas import tpu as pltpu
from jax.experimental.pallas import tpu_sc as plsc

_VOCAB = 1000
_NW = 32
_CHUNK = 64


def _head_table_body(e_ref, w_ref, p_ref):
    p_ref[...] = lax.dot_general(
        e_ref[...], w_ref[...],
        dimension_numbers=(((1,), (1,)), ((), ())),
        preferred_element_type=jnp.float32)


def _head_table(embed_table, head_w_padded):
    v = embed_table.shape[0]
    vp = head_w_padded.shape[0]
    return pl.pallas_call(
        _head_table_body,
        out_shape=jax.ShapeDtypeStruct((v, vp), jnp.float32),
    )(embed_table, head_w_padded)


@functools.partial(jax.jit, static_argnums=(2,))
def _gather_rows(ids, p, n_tokens):
    per_w = n_tokens // _NW
    n_chunks = per_w // _CHUNK
    vp = p.shape[1]
    mesh = plsc.VectorSubcoreMesh(core_axis_name="c", subcore_axis_name="s")

    @functools.partial(
        pl.kernel,
        out_type=jax.ShapeDtypeStruct((n_tokens, _VOCAB), jnp.float32),
        mesh=mesh,
        compiler_params=pltpu.CompilerParams(use_tc_tiling_on_sc=False),
        scratch_types=[
            pltpu.VMEM((per_w,), jnp.int32),
            pltpu.VMEM((_CHUNK, vp), jnp.float32),
            pltpu.SemaphoreType.DMA,
        ],
    )
    def gather(ids_hbm, p_hbm, out_hbm, idx_v, rows_v, sem):
        wid = lax.axis_index("s") * 2 + lax.axis_index("c")
        base = wid * per_w
        pltpu.sync_copy(ids_hbm.at[pl.ds(base, per_w)], idx_v)

        def step(i, carry):
            off = pl.multiple_of(i * _CHUNK, _CHUNK)
            pltpu.async_copy(
                p_hbm.at[idx_v.at[pl.ds(off, _CHUNK)]], rows_v, sem).wait()
            pltpu.sync_copy(rows_v, out_hbm.at[pl.ds(base + off, _CHUNK)])
            return carry

        lax.fori_loop(0, n_chunks, step, 0)

    return gather(ids, p)


def kernel(input_ids, embed_table, head_w):
    b, t = input_ids.shape
    p = _head_table(embed_table, head_w)
    ids = input_ids.reshape(-1).astype(jnp.int32)
    out = _gather_rows(ids, p, b * t)
    return out.reshape(b, t, _VOCAB)
